# baseline (speedup 1.0000x reference)
import functools

import jax
import jax.numpy as jnp
from jax import lax
from jax.experimental import pallas as pl
from jax.experimental.pallas import tpu as pltpu
from jax.experimental.pallas import tpu_sc as plsc

NUM_SRC = 10000
NUM_NEIGH = 16
DIM = 256
LANES = 16

NC = 2
NS = 16
NW = NC * NS

NB = 8
TOTAL_BLOCKS = NUM_SRC // NB
BASE_PER_W = TOTAL_BLOCKS // NW
EXTRA = TOTAL_BLOCKS - BASE_PER_W * NW
MAX_PER_W = BASE_PER_W + 1
HALF_ITERS = (MAX_PER_W + 1) // 2

BLOCK_M = 1000


def _sc_mean_body(neigh, aggr, b0, b1, o0, o1, s0, s1, t0, t1):
    c = lax.axis_index("c")
    s = lax.axis_index("s")
    wid = s * NC + c
    start = wid * BASE_PER_W + jnp.minimum(wid, EXTRA)
    count = BASE_PER_W + (wid < EXTRA).astype(jnp.int32)

    bufs = (b0, b1)
    sems = (s0, s1)
    obufs = (o0, o1)
    osems = (t0, t1)

    def in_copy(j, slot):
        return pltpu.make_async_copy(
            neigh.at[pl.ds((start + j) * NB, NB)], bufs[slot], sems[slot])

    def out_copy(j, slot):
        return pltpu.make_async_copy(
            obufs[slot], aggr.at[pl.ds((start + j) * NB, NB)], osems[slot])

    def compute(slot):
        buf = bufs[slot]
        obuf = obufs[slot]

        def node_body(node, carry):
            for ch in range(DIM // LANES):
                sl = pl.ds(ch * LANES, LANES)
                acc = buf[node, 0, sl]
                for k in range(1, NUM_NEIGH):
                    acc = acc + buf[node, k, sl]
                obuf[node, sl] = acc * (1.0 / NUM_NEIGH)
            return carry

        lax.fori_loop(0, NB, node_body, 0)

    in_copy(0, 0).start()

    def step(jj, carry):
        j0 = 2 * jj
        j1 = j0 + 1
        j2 = j0 + 2

        @pl.when(j0 < count)
        def _():
            in_copy(j0, 0).wait()

        @pl.when(j1 < count)
        def _():
            in_copy(j1, 1).start()

        @pl.when(j0 < count)
        def _():
            @pl.when(j0 >= 2)
            def _():
                out_copy(j0, 0).wait()

            compute(0)
            out_copy(j0, 0).start()

        @pl.when(j1 < count)
        def _():
            in_copy(j1, 1).wait()

        @pl.when(j2 < count)
        def _():
            in_copy(j2, 0).start()

        @pl.when(j1 < count)
        def _():
            @pl.when(j1 >= 2)
            def _():
                out_copy(j1, 1).wait()

            compute(1)
            out_copy(j1, 1).start()

        return carry

    lax.fori_loop(0, HALF_ITERS, step, 0)

    out_copy(0, 0).wait()
    out_copy(0, 1).wait()


def _sc_mean(neigh):
    mesh = plsc.VectorSubcoreMesh(
        core_axis_name="c", subcore_axis_name="s",
        num_cores=NC, num_subcores=NS)
    return pl.kernel(
        _sc_mean_body,
        out_type=jax.ShapeDtypeStruct((NUM_SRC, DIM), jnp.float32),
        mesh=mesh,
        scratch_types=[
            pltpu.VMEM((NB, NUM_NEIGH, DIM), jnp.float32),
            pltpu.VMEM((NB, NUM_NEIGH, DIM), jnp.float32),
            pltpu.VMEM((NB, DIM), jnp.float32),
            pltpu.VMEM((NB, DIM), jnp.float32),
            pltpu.SemaphoreType.DMA,
            pltpu.SemaphoreType.DMA,
            pltpu.SemaphoreType.DMA,
            pltpu.SemaphoreType.DMA,
        ],
    )(neigh)


def _tc_body(aggr_ref, src_ref, w_agg_ref, w_ref, out_ref):
    h = jnp.dot(aggr_ref[...], w_agg_ref[...], preferred_element_type=jnp.float32)
    h = h + src_ref[...]
    out = jnp.dot(h, w_ref[...], preferred_element_type=jnp.float32)
    out_ref[...] = jnp.maximum(out, 0.0)


def _tc_stage(aggr, src, W_agg, W):
    n = src.shape[0]
    return pl.pallas_call(
        _tc_body,
        grid=(n // BLOCK_M,),
        in_specs=[
            pl.BlockSpec((BLOCK_M, DIM), lambda i: (i, 0)),
            pl.BlockSpec((BLOCK_M, DIM), lambda i: (i, 0)),
            pl.BlockSpec((DIM, DIM), lambda i: (0, 0)),
            pl.BlockSpec((DIM, DIM), lambda i: (0, 0)),
        ],
        out_specs=pl.BlockSpec((BLOCK_M, DIM), lambda i: (i, 0)),
        out_shape=jax.ShapeDtypeStruct((n, DIM), jnp.float32),
        compiler_params=pltpu.CompilerParams(
            dimension_semantics=("arbitrary",),
        ),
    )(aggr, src, W_agg, W)


def kernel(src_node_features, nei_node_features, W_agg, W):
    aggr = _sc_mean(nei_node_features)
    return _tc_stage(aggr, src_node_features, W_agg, W)


# speedup vs baseline: 1.8694x; 1.8694x over previous
import jax
import jax.numpy as jnp
from jax import lax
from jax.experimental import pallas as pl
from jax.experimental.pallas import tpu as pltpu
from jax.experimental.pallas import tpu_sc as plsc

NUM_SRC = 10000
NUM_NEIGH = 16
DIM = 256
LANES = 16

NC = 2
NS = 16
NW = NC * NS

NB = 8

SPLIT = 3200
BLK = 400
HEAD_BLOCKS = SPLIT // BLK
TAIL_BLOCKS = (NUM_SRC - SPLIT) // BLK


def _sc_mean_body(neigh, aggr, b0, b1, o0, o1, s0, s1, t0, t1):
    total_blocks = SPLIT // NB
    base_per_w = total_blocks // NW
    extra = total_blocks - base_per_w * NW
    max_per_w = base_per_w + (1 if extra else 0)
    half_iters = (max_per_w + 1) // 2

    c = lax.axis_index("c")
    s = lax.axis_index("s")
    wid = s * NC + c
    start = wid * base_per_w + jnp.minimum(wid, extra)
    count = base_per_w + (wid < extra).astype(jnp.int32)

    bufs = (b0, b1)
    sems = (s0, s1)
    obufs = (o0, o1)
    osems = (t0, t1)

    def in_copy(j, slot):
        return pltpu.make_async_copy(
            neigh.at[pl.ds((start + j) * NB, NB)], bufs[slot], sems[slot])

    def out_copy(j, slot):
        return pltpu.make_async_copy(
            obufs[slot], aggr.at[pl.ds((start + j) * NB, NB)], osems[slot])

    def compute(slot):
        buf = bufs[slot]
        obuf = obufs[slot]

        def node_body(node, carry):
            for ch in range(DIM // LANES):
                sl = pl.ds(ch * LANES, LANES)
                acc = buf[node, 0, sl]
                for k in range(1, NUM_NEIGH):
                    acc = acc + buf[node, k, sl]
                obuf[node, sl] = acc * (1.0 / NUM_NEIGH)
            return carry

        lax.fori_loop(0, NB, node_body, 0)

    in_copy(0, 0).start()

    def step(jj, carry):
        j0 = 2 * jj
        j1 = j0 + 1
        j2 = j0 + 2

        @pl.when(j0 < count)
        def _():
            in_copy(j0, 0).wait()

        @pl.when(j1 < count)
        def _():
            in_copy(j1, 1).start()

        @pl.when(j0 < count)
        def _():
            @pl.when(j0 >= 2)
            def _():
                out_copy(j0, 0).wait()

            compute(0)
            out_copy(j0, 0).start()

        @pl.when(j1 < count)
        def _():
            in_copy(j1, 1).wait()

        @pl.when(j2 < count)
        def _():
            in_copy(j2, 0).start()

        @pl.when(j1 < count)
        def _():
            @pl.when(j1 >= 2)
            def _():
                out_copy(j1, 1).wait()

            compute(1)
            out_copy(j1, 1).start()

        return carry

    lax.fori_loop(0, half_iters, step, 0)

    out_copy(0, 0).wait()
    out_copy(0, 1).wait()


def _sc_mean(neigh):
    mesh = plsc.VectorSubcoreMesh(
        core_axis_name="c", subcore_axis_name="s",
        num_cores=NC, num_subcores=NS)
    return pl.kernel(
        _sc_mean_body,
        out_type=jax.ShapeDtypeStruct((SPLIT, DIM), jnp.float32),
        mesh=mesh,
        scratch_types=[
            pltpu.VMEM((NB, NUM_NEIGH, DIM), jnp.float32),
            pltpu.VMEM((NB, NUM_NEIGH, DIM), jnp.float32),
            pltpu.VMEM((NB, DIM), jnp.float32),
            pltpu.VMEM((NB, DIM), jnp.float32),
            pltpu.SemaphoreType.DMA,
            pltpu.SemaphoreType.DMA,
            pltpu.SemaphoreType.DMA,
            pltpu.SemaphoreType.DMA,
        ],
    )(neigh)


def _fused_body(src_ref, neigh_ref, w_agg_ref, w_ref, out_ref):
    aggr = jnp.sum(neigh_ref[...], axis=1) * (1.0 / NUM_NEIGH)
    h = jnp.dot(aggr, w_agg_ref[...], preferred_element_type=jnp.float32)
    h = h + src_ref[...]
    out = jnp.dot(h, w_ref[...], preferred_element_type=jnp.float32)
    out_ref[...] = jnp.maximum(out, 0.0)


def _tc_fused_tail(src, neigh, W_agg, W):
    return pl.pallas_call(
        _fused_body,
        grid=(TAIL_BLOCKS,),
        in_specs=[
            pl.BlockSpec((BLK, DIM), lambda i: (i + HEAD_BLOCKS, 0)),
            pl.BlockSpec((BLK, NUM_NEIGH, DIM), lambda i: (i + HEAD_BLOCKS, 0, 0)),
            pl.BlockSpec((DIM, DIM), lambda i: (0, 0)),
            pl.BlockSpec((DIM, DIM), lambda i: (0, 0)),
        ],
        out_specs=pl.BlockSpec((BLK, DIM), lambda i: (i + HEAD_BLOCKS, 0)),
        out_shape=jax.ShapeDtypeStruct((NUM_SRC, DIM), jnp.float32),
        compiler_params=pltpu.CompilerParams(
            dimension_semantics=("arbitrary",),
        ),
    )(src, neigh, W_agg, W)


def _finish_body(aggr_ref, src_ref, w_agg_ref, w_ref, _out_alias_ref, out_ref):
    h = jnp.dot(aggr_ref[...], w_agg_ref[...], preferred_element_type=jnp.float32)
    h = h + src_ref[...]
    out = jnp.dot(h, w_ref[...], preferred_element_type=jnp.float32)
    out_ref[...] = jnp.maximum(out, 0.0)


def _tc_finish_head(aggr, src, W_agg, W, partial_out):
    return pl.pallas_call(
        _finish_body,
        grid=(HEAD_BLOCKS,),
        in_specs=[
            pl.BlockSpec((BLK, DIM), lambda i: (i, 0)),
            pl.BlockSpec((BLK, DIM), lambda i: (i, 0)),
            pl.BlockSpec((DIM, DIM), lambda i: (0, 0)),
            pl.BlockSpec((DIM, DIM), lambda i: (0, 0)),
            pl.BlockSpec(memory_space=pl.ANY),
        ],
        out_specs=pl.BlockSpec((BLK, DIM), lambda i: (i, 0)),
        out_shape=jax.ShapeDtypeStruct((NUM_SRC, DIM), jnp.float32),
        input_output_aliases={4: 0},
        compiler_params=pltpu.CompilerParams(
            dimension_semantics=("arbitrary",),
        ),
    )(aggr, src, W_agg, W, partial_out)


def kernel(src_node_features, nei_node_features, W_agg, W):
    aggr_head = _sc_mean(nei_node_features)
    partial = _tc_fused_tail(src_node_features, nei_node_features, W_agg, W)
    return _tc_finish_head(aggr_head, src_node_features, W_agg, W, partial)


# speedup vs baseline: 2.6354x; 1.4098x over previous
import jax
import jax.numpy as jnp
from jax.experimental import pallas as pl
from jax.experimental.pallas import tpu as pltpu

NUM_SRC = 10000
NUM_NEIGH = 16
DIM = 256
BLOCK_M = 1000


def _fused_body(src_ref, neigh_ref, w_agg_ref, w_ref, out_ref):
    aggr = jnp.sum(neigh_ref[...], axis=1) * (1.0 / NUM_NEIGH)
    h = jnp.dot(aggr, w_agg_ref[...], preferred_element_type=jnp.float32)
    h = h + src_ref[...]
    out = jnp.dot(h, w_ref[...], preferred_element_type=jnp.float32)
    out_ref[...] = jnp.maximum(out, 0.0)


def kernel(src_node_features, nei_node_features, W_agg, W):
    n = src_node_features.shape[0]
    grid = (n // BLOCK_M,)
    return pl.pallas_call(
        _fused_body,
        grid=grid,
        in_specs=[
            pl.BlockSpec((BLOCK_M, DIM), lambda i: (i, 0)),
            pl.BlockSpec((BLOCK_M, NUM_NEIGH, DIM), lambda i: (i, 0, 0)),
            pl.BlockSpec((DIM, DIM), lambda i: (0, 0)),
            pl.BlockSpec((DIM, DIM), lambda i: (0, 0)),
        ],
        out_specs=pl.BlockSpec((BLOCK_M, DIM), lambda i: (i, 0)),
        out_shape=jax.ShapeDtypeStruct((n, DIM), jnp.float32),
        compiler_params=pltpu.CompilerParams(
            dimension_semantics=("arbitrary",),
        ),
    )(src_node_features, nei_node_features, W_agg, W)
